# baseline (speedup 1.0000x reference)
import jax
import jax.numpy as jnp
from jax import lax
from jax.experimental import pallas as pl
from jax.experimental.pallas import tpu as pltpu
from jax.experimental.pallas import tpu_sc as plsc

VOCAB = 100000
D = 64
B = 1024
L = 200
M = 4
NHOP = 3

NBT = B // 128
NLG = 32 // NBT
LPW = L // NLG
N_UNITS = NHOP * LPW
G = 128


def _pool_body(story_hbm, table_hbm, out_hbm, idx_v, rows_v, pool_v,
               sem_i0, sem_i1, sem_g0, sem_g1, sem_w0, sem_w1):
    wid = lax.axis_index("s") * 2 + lax.axis_index("c")
    bt = wid % NBT
    l0 = (wid // NBT) * LPW
    b0 = bt * 128
    sem_i = (sem_i0, sem_i1)
    sem_g = (sem_g0, sem_g1)
    sem_w = (sem_w0, sem_w1)

    def unit_info(s):
        s = jnp.minimum(s, N_UNITS - 1)
        h = s // LPW
        l = l0 + s - h * LPW
        return h, l

    def idx_copy(s, buf):
        _, l = unit_info(s)
        return pltpu.make_async_copy(
            story_hbm.at[l, :, pl.ds(b0, 128)], idx_v.at[buf], sem_i[buf])

    def add_offs(s, buf):
        h, _ = unit_info(s)
        off = (h * VOCAB).astype(jnp.int32)
        for j in range(M):
            for k in range(G // 16):
                sl = pl.ds(k * 16, 16)
                idx_v[buf, j, sl] = idx_v[buf, j, sl] + off

    def gather_copies(buf):
        return [pltpu.make_async_copy(
                    table_hbm.at[idx_v.at[buf, j]],
                    rows_v.at[buf, pl.ds(j * G, G)], sem_g[buf])
                for j in range(M)]

    def write_copy(s, buf):
        h, l = unit_info(s)
        return pltpu.make_async_copy(
            pool_v.at[buf], out_hbm.at[h, l, :, bt], sem_w[buf])

    def transpose_sum(buf):
        lane = lax.iota(jnp.int32, 16)
        for g in range(8):
            ridx = [lane + (m * G + g * 16) for m in range(M)]
            sl = pl.ds(g * 16, 16)

            def dbody(dt, c):
                for d8 in range(8):
                    cidx = jnp.full((16,), dt * 8 + d8, jnp.int32)
                    acc = plsc.load_gather(rows_v.at[buf], [ridx[0], cidx])
                    for m in range(1, M):
                        acc = acc + plsc.load_gather(
                            rows_v.at[buf], [ridx[m], cidx])
                    pool_v[buf, dt, d8, sl] = acc
                return c

            lax.fori_loop(0, 8, dbody, 0)

    idx_copy(0, 0).start()
    idx_copy(1, 1).start()
    idx_copy(0, 0).wait()
    add_offs(0, 0)
    for c in gather_copies(0):
        c.start()

    def phase(s, buf):
        nxt = 1 - buf
        idx_copy(s + 1, nxt).wait()
        add_offs(s + 1, nxt)
        for c in gather_copies(nxt):
            c.start()
        for c in gather_copies(buf):
            c.wait()
        idx_copy(s + 2, buf).start()

        @pl.when(s >= 2)
        def _():
            write_copy(s - 2, buf).wait()

        transpose_sum(buf)
        write_copy(s, buf).start()

    def loop_body(t, c):
        phase(2 * t, 0)
        phase(2 * t + 1, 1)
        return c
    lax.fori_loop(0, N_UNITS // 2, loop_body, 0)

    for c in gather_copies(0):
        c.wait()
    idx_copy(N_UNITS + 1, 1).wait()
    write_copy(N_UNITS - 2, 0).wait()
    write_copy(N_UNITS - 1, 1).wait()


def _pool_sc(story_t, table):
    f = pl.kernel(
        _pool_body,
        out_type=jax.ShapeDtypeStruct((NHOP, L, 8, NBT, 8, 128),
                                      jnp.float32),
        mesh=plsc.VectorSubcoreMesh(core_axis_name="c", subcore_axis_name="s",
                                    num_cores=2, num_subcores=16),
        scratch_types=[
            pltpu.VMEM((2, M, G), jnp.int32),
            pltpu.VMEM((2, M * G, D), jnp.float32),
            pltpu.VMEM((2, 8, 8, 128), jnp.float32),
            pltpu.SemaphoreType.DMA,
            pltpu.SemaphoreType.DMA,
            pltpu.SemaphoreType.DMA,
            pltpu.SemaphoreType.DMA,
            pltpu.SemaphoreType.DMA,
            pltpu.SemaphoreType.DMA,
        ],
        compiler_params=pltpu.CompilerParams(use_tc_tiling_on_sc=False,
                                             needs_layout_passes=False),
    )
    return f(story_t, table)


LCH = 40
NCH = L // LCH


def _attn_body(q_ref, gp_ref, dh_ref, len_ref, p_ref, soft_ref, logit_ref,
               pbuf, sem0, sem1):
    bt = pl.program_id(0)
    sems = (sem0, sem1)

    def hop_copy(h):
        return pltpu.make_async_copy(
            p_ref.at[h, :, :, bt, :, :], pbuf.at[h % 2], sems[h % 2])

    hop_copy(0).start()
    hop_copy(1).start()

    u = q_ref[...]
    lens = len_ref[...][0:1, :]
    gp = gp_ref[...]

    def mem_slice(h, c):
        lsl = pl.ds(c * LCH, LCH)
        lpos = lax.broadcasted_iota(jnp.int32, (LCH, 128), 0) + c * LCH
        mask = (lpos < lens).astype(jnp.float32)
        p = pbuf[h % 2, lsl, :, :, :].reshape(LCH, D, 128)
        return p + dh_ref[lsl, :, :] * mask[:, None, :]

    logits = None
    soft = None
    for h in range(NHOP):
        if h == 0:
            hop_copy(0).wait()
        parts = [jnp.sum(mem_slice(h, c) * u[None, :, :], axis=1)
                 for c in range(NCH)]
        logits = jnp.concatenate(parts, axis=0) * gp
        mx = jnp.max(logits, axis=0, keepdims=True)
        e = jnp.exp(logits - mx)
        soft = e / jnp.sum(e, axis=0, keepdims=True)
        if h < NHOP - 1:
            hop_copy(h + 1).wait()
            if h + 2 <= NHOP - 1:
                hop_copy(h + 2).start()
            w = soft * gp
            o = jnp.zeros((D, 128), jnp.float32)
            for c in range(NCH):
                wc = w[c * LCH:(c + 1) * LCH, None, :]
                o = o + jnp.sum(mem_slice(h + 1, c) * wc, axis=0)
            u = u + o
    soft_ref[...] = soft
    logit_ref[...] = logits


def _attn_tc(q_t, gp_t, dh_t, lens8, pooled6):
    grid = (NBT,)
    out_shape = [
        jax.ShapeDtypeStruct((L, B), jnp.float32),
        jax.ShapeDtypeStruct((L, B), jnp.float32),
    ]
    return pl.pallas_call(
        _attn_body,
        grid=grid,
        in_specs=[
            pl.BlockSpec((D, 128), lambda i: (0, i)),
            pl.BlockSpec((L, 128), lambda i: (0, i)),
            pl.BlockSpec((L, D, 128), lambda i: (0, 0, i)),
            pl.BlockSpec((8, 128), lambda i: (0, i)),
            pl.BlockSpec(memory_space=pl.ANY),
        ],
        out_specs=[
            pl.BlockSpec((L, 128), lambda i: (0, i)),
            pl.BlockSpec((L, 128), lambda i: (0, i)),
        ],
        out_shape=out_shape,
        scratch_shapes=[
            pltpu.VMEM((2, L, 8, 8, 128), jnp.float32),
            pltpu.SemaphoreType.DMA,
            pltpu.SemaphoreType.DMA,
        ],
    )(q_t, gp_t, dh_t, lens8, pooled6)


def kernel(query_vector, global_pointer, dh_outputs, CS, story, conv_len):
    table = CS.reshape(-1, D)
    story_t = jnp.transpose(story, (1, 2, 0)).astype(jnp.int32)
    dh_t = jnp.transpose(dh_outputs, (1, 2, 0))
    gp_t = global_pointer.T
    q_t = query_vector.T
    lens8 = jnp.broadcast_to(conv_len.reshape(1, B).astype(jnp.int32), (8, B))
    pooled6 = _pool_sc(story_t, table)
    soft_t, logits_t = _attn_tc(q_t, gp_t, dh_t, lens8, pooled6)
    return (soft_t.T, logits_t.T)


# speedup vs baseline: 1.8924x; 1.8924x over previous
import jax
import jax.numpy as jnp
from jax import lax
from jax.experimental import pallas as pl
from jax.experimental.pallas import tpu as pltpu
from jax.experimental.pallas import tpu_sc as plsc

VOCAB = 100000
D = 64
B = 1024
L = 200
M = 4
NHOP = 3

NBT = B // 128
NLG = 32 // NBT
LPW = L // NLG
N_UNITS = NHOP * LPW
G = 128


def _pool_body(story_hbm, table_hbm, out_hbm, idx_v, rows_v, pool_v,
               sem_i0, sem_i1, sem_g0, sem_g1, sem_w0, sem_w1):
    wid = lax.axis_index("s") * 2 + lax.axis_index("c")
    bt = wid % NBT
    l0 = (wid // NBT) * LPW
    b0 = bt * 128
    sem_i = (sem_i0, sem_i1)
    sem_g = (sem_g0, sem_g1)
    sem_w = (sem_w0, sem_w1)

    def unit_info(s):
        s = jnp.minimum(s, N_UNITS - 1)
        h = s // LPW
        l = l0 + s - h * LPW
        return h, l

    def idx_copy(s, buf):
        _, l = unit_info(s)
        return pltpu.make_async_copy(
            story_hbm.at[l, :, pl.ds(b0, 128)], idx_v.at[buf], sem_i[buf])

    def add_offs(s, buf):
        h, _ = unit_info(s)
        off = (h * VOCAB).astype(jnp.int32)
        for j in range(M):
            for k in range(G // 16):
                sl = pl.ds(k * 16, 16)
                idx_v[buf, j, sl] = idx_v[buf, j, sl] + off

    def gather_copies(buf):
        return [pltpu.make_async_copy(
                    table_hbm.at[idx_v.at[buf, j]],
                    rows_v.at[buf, pl.ds(j * G, G)], sem_g[buf])
                for j in range(M)]

    def write_copy(s, buf):
        h, l = unit_info(s)
        return pltpu.make_async_copy(
            pool_v.at[buf, :, :, pl.ds(0, 128)], out_hbm.at[h, l, :, bt],
            sem_w[buf])

    def transpose_sum(buf):
        lane = lax.iota(jnp.int32, 16)
        dtv = [(lane + g * 16) // 8 for g in range(D // 16)]
        d8v = [(lane + g * 16) % 8 for g in range(D // 16)]
        for m in range(M):
            def rbody(bl, c):
                r = m * G + bl
                bv = jnp.full((16,), bl, jnp.int32)
                for g in range(D // 16):
                    v = rows_v[buf, r, pl.ds(g * 16, 16)]
                    if m == 0:
                        plsc.store_scatter(
                            pool_v.at[buf], [dtv[g], d8v[g], bv], v)
                    else:
                        plsc.addupdate_scatter(
                            pool_v.at[buf], [dtv[g], d8v[g], bv], v)
                return c
            lax.fori_loop(0, G, rbody, 0)

    idx_copy(0, 0).start()
    idx_copy(1, 1).start()
    idx_copy(0, 0).wait()
    add_offs(0, 0)
    for c in gather_copies(0):
        c.start()

    def phase(s, buf):
        nxt = 1 - buf
        idx_copy(s + 1, nxt).wait()
        add_offs(s + 1, nxt)
        for c in gather_copies(nxt):
            c.start()
        for c in gather_copies(buf):
            c.wait()
        idx_copy(s + 2, buf).start()

        @pl.when(s >= 2)
        def _():
            write_copy(s - 2, buf).wait()

        transpose_sum(buf)
        write_copy(s, buf).start()

    def loop_body(t, c):
        phase(2 * t, 0)
        phase(2 * t + 1, 1)
        return c
    lax.fori_loop(0, N_UNITS // 2, loop_body, 0)

    for c in gather_copies(0):
        c.wait()
    idx_copy(N_UNITS + 1, 1).wait()
    write_copy(N_UNITS - 2, 0).wait()
    write_copy(N_UNITS - 1, 1).wait()


def _pool_sc(story_t, table):
    f = pl.kernel(
        _pool_body,
        out_type=jax.ShapeDtypeStruct((NHOP, L, 8, NBT, 8, 128),
                                      jnp.float32),
        mesh=plsc.VectorSubcoreMesh(core_axis_name="c", subcore_axis_name="s",
                                    num_cores=2, num_subcores=16),
        scratch_types=[
            pltpu.VMEM((2, M, G), jnp.int32),
            pltpu.VMEM((2, M * G, D), jnp.float32),
            pltpu.VMEM((2, 8, 8, 129), jnp.float32),
            pltpu.SemaphoreType.DMA,
            pltpu.SemaphoreType.DMA,
            pltpu.SemaphoreType.DMA,
            pltpu.SemaphoreType.DMA,
            pltpu.SemaphoreType.DMA,
            pltpu.SemaphoreType.DMA,
        ],
        compiler_params=pltpu.CompilerParams(use_tc_tiling_on_sc=False,
                                             needs_layout_passes=False),
    )
    return f(story_t, table)


LCH = 40
NCH = L // LCH


def _attn_body(q_ref, gp_ref, dh_ref, len_ref, p_ref, soft_ref, logit_ref,
               pbuf, sem0, sem1):
    bt = pl.program_id(0)
    sems = (sem0, sem1)

    def hop_copy(h):
        return pltpu.make_async_copy(
            p_ref.at[h, :, :, bt, :, :], pbuf.at[h % 2], sems[h % 2])

    hop_copy(0).start()
    hop_copy(1).start()

    u = q_ref[...]
    lens = len_ref[...][0:1, :]
    gp = gp_ref[...]

    def mem_slice(h, c):
        lsl = pl.ds(c * LCH, LCH)
        lpos = lax.broadcasted_iota(jnp.int32, (LCH, 128), 0) + c * LCH
        mask = (lpos < lens).astype(jnp.float32)
        p = pbuf[h % 2, lsl, :, :, :].reshape(LCH, D, 128)
        return p + dh_ref[lsl, :, :] * mask[:, None, :]

    logits = None
    soft = None
    for h in range(NHOP):
        if h == 0:
            hop_copy(0).wait()
        parts = [jnp.sum(mem_slice(h, c) * u[None, :, :], axis=1)
                 for c in range(NCH)]
        logits = jnp.concatenate(parts, axis=0) * gp
        mx = jnp.max(logits, axis=0, keepdims=True)
        e = jnp.exp(logits - mx)
        soft = e / jnp.sum(e, axis=0, keepdims=True)
        if h < NHOP - 1:
            hop_copy(h + 1).wait()
            if h + 2 <= NHOP - 1:
                hop_copy(h + 2).start()
            w = soft * gp
            o = jnp.zeros((D, 128), jnp.float32)
            for c in range(NCH):
                wc = w[c * LCH:(c + 1) * LCH, None, :]
                o = o + jnp.sum(mem_slice(h + 1, c) * wc, axis=0)
            u = u + o
    soft_ref[...] = soft
    logit_ref[...] = logits


def _attn_tc(q_t, gp_t, dh_t, lens8, pooled6):
    grid = (NBT,)
    out_shape = [
        jax.ShapeDtypeStruct((L, B), jnp.float32),
        jax.ShapeDtypeStruct((L, B), jnp.float32),
    ]
    return pl.pallas_call(
        _attn_body,
        grid=grid,
        in_specs=[
            pl.BlockSpec((D, 128), lambda i: (0, i)),
            pl.BlockSpec((L, 128), lambda i: (0, i)),
            pl.BlockSpec((L, D, 128), lambda i: (0, 0, i)),
            pl.BlockSpec((8, 128), lambda i: (0, i)),
            pl.BlockSpec(memory_space=pl.ANY),
        ],
        out_specs=[
            pl.BlockSpec((L, 128), lambda i: (0, i)),
            pl.BlockSpec((L, 128), lambda i: (0, i)),
        ],
        out_shape=out_shape,
        scratch_shapes=[
            pltpu.VMEM((2, L, 8, 8, 128), jnp.float32),
            pltpu.SemaphoreType.DMA,
            pltpu.SemaphoreType.DMA,
        ],
    )(q_t, gp_t, dh_t, lens8, pooled6)


def kernel(query_vector, global_pointer, dh_outputs, CS, story, conv_len):
    table = CS.reshape(-1, D)
    story_t = jnp.transpose(story, (1, 2, 0)).astype(jnp.int32)
    dh_t = jnp.transpose(dh_outputs, (1, 2, 0))
    gp_t = global_pointer.T
    q_t = query_vector.T
    lens8 = jnp.broadcast_to(conv_len.reshape(1, B).astype(jnp.int32), (8, B))
    pooled6 = _pool_sc(story_t, table)
    soft_t, logits_t = _attn_tc(q_t, gp_t, dh_t, lens8, pooled6)
    return (soft_t.T, logits_t.T)


# speedup vs baseline: 1.9194x; 1.0143x over previous
import jax
import jax.numpy as jnp
from jax import lax
from jax.experimental import pallas as pl
from jax.experimental.pallas import tpu as pltpu
from jax.experimental.pallas import tpu_sc as plsc

VOCAB = 100000
D = 64
B = 1024
L = 200
M = 4
NHOP = 3

NBT = B // 128
NLG = 32 // NBT
LPW = L // NLG
N_UNITS = NHOP * LPW
G = 128


def _pool_body(story_hbm, table_hbm, out_hbm, idx_v, rows_v, pool_v,
               sem_i0, sem_i1, sem_g0, sem_g1, sem_w0, sem_w1):
    wid = lax.axis_index("s") * 2 + lax.axis_index("c")
    bt = wid % NBT
    l0 = (wid // NBT) * LPW
    b0 = bt * 128
    sem_i = (sem_i0, sem_i1)
    sem_g = (sem_g0, sem_g1)
    sem_w = (sem_w0, sem_w1)

    def unit_info(s):
        s = jnp.minimum(s, N_UNITS - 1)
        h = s // LPW
        l = l0 + s - h * LPW
        return h, l

    def idx_copy(s, buf):
        _, l = unit_info(s)
        return pltpu.make_async_copy(
            story_hbm.at[l, :, pl.ds(b0, 128)], idx_v.at[buf], sem_i[buf])

    def add_offs(s, buf):
        h, _ = unit_info(s)
        off = (h * VOCAB).astype(jnp.int32)
        for j in range(M):
            for k in range(G // 16):
                sl = pl.ds(k * 16, 16)
                idx_v[buf, j, sl] = idx_v[buf, j, sl] + off

    def gather_copies(buf):
        return [pltpu.make_async_copy(
                    table_hbm.at[idx_v.at[buf, j]],
                    rows_v.at[buf, pl.ds(j * G, G)], sem_g[buf])
                for j in range(M)]

    def write_copy(s, buf):
        h, l = unit_info(s)
        return pltpu.make_async_copy(
            pool_v.at[buf, :, :, pl.ds(0, 128)], out_hbm.at[h, l, :, bt],
            sem_w[buf])

    def transpose_sum(buf):
        lane = lax.iota(jnp.int32, 16)
        dtv = [(lane + g * 16) // 8 for g in range(D // 16)]
        d8v = [(lane + g * 16) % 8 for g in range(D // 16)]
        for m in range(M):
            def rbody(bl, c):
                r = m * G + bl
                bv = jnp.full((16,), bl, jnp.int32)
                for g in range(D // 16):
                    v = rows_v[buf, r, pl.ds(g * 16, 16)]
                    if m == 0:
                        plsc.store_scatter(
                            pool_v.at[buf], [dtv[g], d8v[g], bv], v)
                    else:
                        plsc.addupdate_scatter(
                            pool_v.at[buf], [dtv[g], d8v[g], bv], v)
                return c
            lax.fori_loop(0, G, rbody, 0, unroll=8)

    idx_copy(0, 0).start()
    idx_copy(1, 1).start()
    idx_copy(0, 0).wait()
    add_offs(0, 0)
    for c in gather_copies(0):
        c.start()

    def phase(s, buf):
        nxt = 1 - buf
        idx_copy(s + 1, nxt).wait()
        add_offs(s + 1, nxt)
        for c in gather_copies(nxt):
            c.start()
        for c in gather_copies(buf):
            c.wait()
        idx_copy(s + 2, buf).start()

        @pl.when(s >= 2)
        def _():
            write_copy(s - 2, buf).wait()

        transpose_sum(buf)
        write_copy(s, buf).start()

    def loop_body(t, c):
        phase(2 * t, 0)
        phase(2 * t + 1, 1)
        return c
    lax.fori_loop(0, N_UNITS // 2, loop_body, 0)

    for c in gather_copies(0):
        c.wait()
    idx_copy(N_UNITS + 1, 1).wait()
    write_copy(N_UNITS - 2, 0).wait()
    write_copy(N_UNITS - 1, 1).wait()


def _pool_sc(story_t, table):
    f = pl.kernel(
        _pool_body,
        out_type=jax.ShapeDtypeStruct((NHOP, L, 8, NBT, 8, 128),
                                      jnp.float32),
        mesh=plsc.VectorSubcoreMesh(core_axis_name="c", subcore_axis_name="s",
                                    num_cores=2, num_subcores=16),
        scratch_types=[
            pltpu.VMEM((2, M, G), jnp.int32),
            pltpu.VMEM((2, M * G, D), jnp.float32),
            pltpu.VMEM((2, 8, 8, 129), jnp.float32),
            pltpu.SemaphoreType.DMA,
            pltpu.SemaphoreType.DMA,
            pltpu.SemaphoreType.DMA,
            pltpu.SemaphoreType.DMA,
            pltpu.SemaphoreType.DMA,
            pltpu.SemaphoreType.DMA,
        ],
        compiler_params=pltpu.CompilerParams(use_tc_tiling_on_sc=False,
                                             needs_layout_passes=False),
    )
    return f(story_t, table)


LCH = 40
NCH = L // LCH


def _attn_body(q_ref, gp_ref, dh_ref, len_ref, p_ref, soft_ref, logit_ref,
               pbuf, sem0, sem1):
    bt = pl.program_id(0)
    sems = (sem0, sem1)

    def hop_copy(h):
        return pltpu.make_async_copy(
            p_ref.at[h, :, :, bt, :, :], pbuf.at[h % 2], sems[h % 2])

    hop_copy(0).start()
    hop_copy(1).start()

    u = q_ref[...]
    lens = len_ref[...][0:1, :]
    gp = gp_ref[...]

    def mem_slice(h, c):
        lsl = pl.ds(c * LCH, LCH)
        lpos = lax.broadcasted_iota(jnp.int32, (LCH, 128), 0) + c * LCH
        mask = (lpos < lens).astype(jnp.float32)
        p = pbuf[h % 2, lsl, :, :, :].reshape(LCH, D, 128)
        return p + dh_ref[lsl, :, :] * mask[:, None, :]

    logits = None
    soft = None
    for h in range(NHOP):
        if h == 0:
            hop_copy(0).wait()
        parts = [jnp.sum(mem_slice(h, c) * u[None, :, :], axis=1)
                 for c in range(NCH)]
        logits = jnp.concatenate(parts, axis=0) * gp
        mx = jnp.max(logits, axis=0, keepdims=True)
        e = jnp.exp(logits - mx)
        soft = e / jnp.sum(e, axis=0, keepdims=True)
        if h < NHOP - 1:
            hop_copy(h + 1).wait()
            if h + 2 <= NHOP - 1:
                hop_copy(h + 2).start()
            w = soft * gp
            o = jnp.zeros((D, 128), jnp.float32)
            for c in range(NCH):
                wc = w[c * LCH:(c + 1) * LCH, None, :]
                o = o + jnp.sum(mem_slice(h + 1, c) * wc, axis=0)
            u = u + o
    soft_ref[...] = soft
    logit_ref[...] = logits


def _attn_tc(q_t, gp_t, dh_t, lens8, pooled6):
    grid = (NBT,)
    out_shape = [
        jax.ShapeDtypeStruct((L, B), jnp.float32),
        jax.ShapeDtypeStruct((L, B), jnp.float32),
    ]
    return pl.pallas_call(
        _attn_body,
        grid=grid,
        in_specs=[
            pl.BlockSpec((D, 128), lambda i: (0, i)),
            pl.BlockSpec((L, 128), lambda i: (0, i)),
            pl.BlockSpec((L, D, 128), lambda i: (0, 0, i)),
            pl.BlockSpec((8, 128), lambda i: (0, i)),
            pl.BlockSpec(memory_space=pl.ANY),
        ],
        out_specs=[
            pl.BlockSpec((L, 128), lambda i: (0, i)),
            pl.BlockSpec((L, 128), lambda i: (0, i)),
        ],
        out_shape=out_shape,
        scratch_shapes=[
            pltpu.VMEM((2, L, 8, 8, 128), jnp.float32),
            pltpu.SemaphoreType.DMA,
            pltpu.SemaphoreType.DMA,
        ],
    )(q_t, gp_t, dh_t, lens8, pooled6)


def kernel(query_vector, global_pointer, dh_outputs, CS, story, conv_len):
    table = CS.reshape(-1, D)
    story_t = jnp.transpose(story, (1, 2, 0)).astype(jnp.int32)
    dh_t = jnp.transpose(dh_outputs, (1, 2, 0))
    gp_t = global_pointer.T
    q_t = query_vector.T
    lens8 = jnp.broadcast_to(conv_len.reshape(1, B).astype(jnp.int32), (8, B))
    pooled6 = _pool_sc(story_t, table)
    soft_t, logits_t = _attn_tc(q_t, gp_t, dh_t, lens8, pooled6)
    return (soft_t.T, logits_t.T)


# speedup vs baseline: 3.6964x; 1.9258x over previous
import jax
import jax.numpy as jnp
from jax import lax
from jax.experimental import pallas as pl
from jax.experimental.pallas import tpu as pltpu
from jax.experimental.pallas import tpu_sc as plsc

VOCAB = 100000
D = 64
B = 1024
L = 200
M = 4
NHOP = 3

NBT = B // 128
NLG = 32 // NBT
LPW = L // NLG
N_UNITS = NHOP * LPW
G = 128


def _pool_body(story_hbm, table_hbm, out_hbm, idx_v, rows_v, pool_v,
               sem_i0, sem_i1, sem_g0, sem_g1, sem_w0, sem_w1):
    wid = lax.axis_index("s") * 2 + lax.axis_index("c")
    bt = wid % NBT
    l0 = (wid // NBT) * LPW
    b0 = bt * 128
    sem_i = (sem_i0, sem_i1)
    sem_g = (sem_g0, sem_g1)
    sem_w = (sem_w0, sem_w1)

    def unit_info(s):
        s = jnp.minimum(s, N_UNITS - 1)
        h = s // LPW
        l = l0 + s - h * LPW
        return h, l

    def idx_copy(s, buf):
        _, l = unit_info(s)
        return pltpu.make_async_copy(
            story_hbm.at[l, :, pl.ds(b0, 128)], idx_v.at[buf], sem_i[buf])

    def add_offs(s, buf):
        h, _ = unit_info(s)
        off = (h * VOCAB).astype(jnp.int32)
        for j in range(M):
            for k in range(G // 16):
                sl = pl.ds(k * 16, 16)
                idx_v[buf, j, sl] = idx_v[buf, j, sl] + off

    def gather_copies(buf):
        return [pltpu.make_async_copy(
                    table_hbm.at[idx_v.at[buf, j]],
                    rows_v.at[buf, pl.ds(j * G, G)], sem_g[buf])
                for j in range(M)]

    def write_copy(s, buf):
        h, l = unit_info(s)
        return pltpu.make_async_copy(
            pool_v.at[buf, :, :, pl.ds(0, 128)], out_hbm.at[h, l, :, bt],
            sem_w[buf])

    def transpose_sum(buf):
        lane = lax.iota(jnp.int32, 16)
        dtv = [(lane + g * 16) // 8 for g in range(D // 16)]
        d8v = [(lane + g * 16) % 8 for g in range(D // 16)]

        def rbody(bl, c):
            bv = jnp.full((16,), bl, jnp.int32)
            for g in range(D // 16):
                sl = pl.ds(g * 16, 16)
                s = (rows_v[buf, bl, sl] + rows_v[buf, G + bl, sl]
                     + rows_v[buf, 2 * G + bl, sl]
                     + rows_v[buf, 3 * G + bl, sl])
                plsc.store_scatter(pool_v.at[buf], [dtv[g], d8v[g], bv], s)
            return c
        lax.fori_loop(0, G, rbody, 0, unroll=4)

    idx_copy(0, 0).start()
    idx_copy(1, 1).start()
    idx_copy(0, 0).wait()
    add_offs(0, 0)
    for c in gather_copies(0):
        c.start()

    def phase(s, buf):
        nxt = 1 - buf
        idx_copy(s + 1, nxt).wait()
        add_offs(s + 1, nxt)
        for c in gather_copies(nxt):
            c.start()
        for c in gather_copies(buf):
            c.wait()
        idx_copy(s + 2, buf).start()

        @pl.when(s >= 2)
        def _():
            write_copy(s - 2, buf).wait()

        transpose_sum(buf)
        write_copy(s, buf).start()

    def loop_body(t, c):
        phase(2 * t, 0)
        phase(2 * t + 1, 1)
        return c
    lax.fori_loop(0, N_UNITS // 2, loop_body, 0)

    for c in gather_copies(0):
        c.wait()
    idx_copy(N_UNITS + 1, 1).wait()
    write_copy(N_UNITS - 2, 0).wait()
    write_copy(N_UNITS - 1, 1).wait()


def _pool_sc(story_t, table):
    f = pl.kernel(
        _pool_body,
        out_type=jax.ShapeDtypeStruct((NHOP, L, 8, NBT, 8, 128),
                                      jnp.float32),
        mesh=plsc.VectorSubcoreMesh(core_axis_name="c", subcore_axis_name="s",
                                    num_cores=2, num_subcores=16),
        scratch_types=[
            pltpu.VMEM((2, M, G), jnp.int32),
            pltpu.VMEM((2, M * G, D), jnp.float32),
            pltpu.VMEM((2, 8, 8, 129), jnp.float32),
            pltpu.SemaphoreType.DMA,
            pltpu.SemaphoreType.DMA,
            pltpu.SemaphoreType.DMA,
            pltpu.SemaphoreType.DMA,
            pltpu.SemaphoreType.DMA,
            pltpu.SemaphoreType.DMA,
        ],
        compiler_params=pltpu.CompilerParams(use_tc_tiling_on_sc=False,
                                             needs_layout_passes=False),
    )
    return f(story_t, table)


LCH = 40
NCH = L // LCH


def _attn_body(q_ref, gp_ref, dh_ref, len_ref, p_ref, soft_ref, logit_ref,
               pbuf, sem0, sem1):
    bt = pl.program_id(0)
    sems = (sem0, sem1)

    def hop_copy(h):
        return pltpu.make_async_copy(
            p_ref.at[h, :, :, bt, :, :], pbuf.at[h % 2], sems[h % 2])

    hop_copy(0).start()
    hop_copy(1).start()

    u = q_ref[...]
    lens = len_ref[...][0:1, :]
    gp = gp_ref[...]

    def mem_slice(h, c):
        lsl = pl.ds(c * LCH, LCH)
        lpos = lax.broadcasted_iota(jnp.int32, (LCH, 128), 0) + c * LCH
        mask = (lpos < lens).astype(jnp.float32)
        p = pbuf[h % 2, lsl, :, :, :].reshape(LCH, D, 128)
        return p + dh_ref[lsl, :, :] * mask[:, None, :]

    logits = None
    soft = None
    for h in range(NHOP):
        if h == 0:
            hop_copy(0).wait()
        parts = [jnp.sum(mem_slice(h, c) * u[None, :, :], axis=1)
                 for c in range(NCH)]
        logits = jnp.concatenate(parts, axis=0) * gp
        mx = jnp.max(logits, axis=0, keepdims=True)
        e = jnp.exp(logits - mx)
        soft = e / jnp.sum(e, axis=0, keepdims=True)
        if h < NHOP - 1:
            hop_copy(h + 1).wait()
            if h + 2 <= NHOP - 1:
                hop_copy(h + 2).start()
            w = soft * gp
            o = jnp.zeros((D, 128), jnp.float32)
            for c in range(NCH):
                wc = w[c * LCH:(c + 1) * LCH, None, :]
                o = o + jnp.sum(mem_slice(h + 1, c) * wc, axis=0)
            u = u + o
    soft_ref[...] = soft
    logit_ref[...] = logits


def _attn_tc(q_t, gp_t, dh_t, lens8, pooled6):
    grid = (NBT,)
    out_shape = [
        jax.ShapeDtypeStruct((L, B), jnp.float32),
        jax.ShapeDtypeStruct((L, B), jnp.float32),
    ]
    return pl.pallas_call(
        _attn_body,
        grid=grid,
        in_specs=[
            pl.BlockSpec((D, 128), lambda i: (0, i)),
            pl.BlockSpec((L, 128), lambda i: (0, i)),
            pl.BlockSpec((L, D, 128), lambda i: (0, 0, i)),
            pl.BlockSpec((8, 128), lambda i: (0, i)),
            pl.BlockSpec(memory_space=pl.ANY),
        ],
        out_specs=[
            pl.BlockSpec((L, 128), lambda i: (0, i)),
            pl.BlockSpec((L, 128), lambda i: (0, i)),
        ],
        out_shape=out_shape,
        scratch_shapes=[
            pltpu.VMEM((2, L, 8, 8, 128), jnp.float32),
            pltpu.SemaphoreType.DMA,
            pltpu.SemaphoreType.DMA,
        ],
    )(q_t, gp_t, dh_t, lens8, pooled6)


def kernel(query_vector, global_pointer, dh_outputs, CS, story, conv_len):
    table = CS.reshape(-1, D)
    story_t = jnp.transpose(story, (1, 2, 0)).astype(jnp.int32)
    dh_t = jnp.transpose(dh_outputs, (1, 2, 0))
    gp_t = global_pointer.T
    q_t = query_vector.T
    lens8 = jnp.broadcast_to(conv_len.reshape(1, B).astype(jnp.int32), (8, B))
    pooled6 = _pool_sc(story_t, table)
    soft_t, logits_t = _attn_tc(q_t, gp_t, dh_t, lens8, pooled6)
    return (soft_t.T, logits_t.T)


# speedup vs baseline: 3.7234x; 1.0073x over previous
import jax
import jax.numpy as jnp
from jax import lax
from jax.experimental import pallas as pl
from jax.experimental.pallas import tpu as pltpu
from jax.experimental.pallas import tpu_sc as plsc

VOCAB = 100000
D = 64
B = 1024
L = 200
M = 4
NHOP = 3

NBT = B // 128
NLG = 32 // NBT
LPW = L // NLG
N_UNITS = NHOP * LPW
G = 128


def _pool_body(story_hbm, table_hbm, out_hbm, idx_v, rows_v, pool_v,
               sem_i0, sem_i1, sem_g0, sem_g1, sem_w0, sem_w1):
    wid = lax.axis_index("s") * 2 + lax.axis_index("c")
    bt = wid % NBT
    l0 = (wid // NBT) * LPW
    b0 = bt * 128
    sem_i = (sem_i0, sem_i1)
    sem_g = (sem_g0, sem_g1)
    sem_w = (sem_w0, sem_w1)

    def unit_info(s):
        s = jnp.minimum(s, N_UNITS - 1)
        h = s // LPW
        l = l0 + s - h * LPW
        return h, l

    def idx_copy(s, buf):
        _, l = unit_info(s)
        return pltpu.make_async_copy(
            story_hbm.at[l, :, pl.ds(b0, 128)], idx_v.at[buf], sem_i[buf])

    def add_offs(s, buf):
        h, _ = unit_info(s)
        off = (h * VOCAB).astype(jnp.int32)
        for j in range(M):
            for k in range(G // 16):
                sl = pl.ds(k * 16, 16)
                idx_v[buf, j, sl] = idx_v[buf, j, sl] + off

    def gather_copies(buf):
        return [pltpu.make_async_copy(
                    table_hbm.at[idx_v.at[buf, j]],
                    rows_v.at[buf, pl.ds(j * G, G)], sem_g[buf])
                for j in range(M)]

    def write_copy(s, buf):
        h, l = unit_info(s)
        return pltpu.make_async_copy(
            pool_v.at[buf, :, :, pl.ds(0, 128)], out_hbm.at[h, l, :, bt],
            sem_w[buf])

    def transpose_sum(buf):
        lane = lax.iota(jnp.int32, 16)
        dtv = [(lane + g * 16) // 8 for g in range(D // 16)]
        d8v = [(lane + g * 16) % 8 for g in range(D // 16)]

        def rbody(bl, c):
            bv = jnp.full((16,), bl, jnp.int32)
            for g in range(D // 16):
                sl = pl.ds(g * 16, 16)
                s = (rows_v[buf, bl, sl] + rows_v[buf, G + bl, sl]
                     + rows_v[buf, 2 * G + bl, sl]
                     + rows_v[buf, 3 * G + bl, sl])
                plsc.store_scatter(pool_v.at[buf], [dtv[g], d8v[g], bv], s)
            return c
        lax.fori_loop(0, G, rbody, 0, unroll=4)

    idx_copy(0, 0).start()
    idx_copy(1, 1).start()
    idx_copy(0, 0).wait()
    add_offs(0, 0)
    for c in gather_copies(0):
        c.start()

    def phase(s, buf):
        nxt = 1 - buf
        idx_copy(s + 1, nxt).wait()
        add_offs(s + 1, nxt)
        for c in gather_copies(nxt):
            c.start()
        for c in gather_copies(buf):
            c.wait()
        idx_copy(s + 2, buf).start()

        @pl.when(s >= 2)
        def _():
            write_copy(s - 2, buf).wait()

        transpose_sum(buf)
        write_copy(s, buf).start()

    def loop_body(t, c):
        phase(2 * t, 0)
        phase(2 * t + 1, 1)
        return c
    lax.fori_loop(0, N_UNITS // 2, loop_body, 0)

    for c in gather_copies(0):
        c.wait()
    idx_copy(N_UNITS + 1, 1).wait()
    write_copy(N_UNITS - 2, 0).wait()
    write_copy(N_UNITS - 1, 1).wait()


def _pool_sc(story_t, table):
    f = pl.kernel(
        _pool_body,
        out_type=jax.ShapeDtypeStruct((NHOP, L, 8, NBT, 8, 128),
                                      jnp.float32),
        mesh=plsc.VectorSubcoreMesh(core_axis_name="c", subcore_axis_name="s",
                                    num_cores=2, num_subcores=16),
        scratch_types=[
            pltpu.VMEM((2, M, G), jnp.int32),
            pltpu.VMEM((2, M * G, D), jnp.float32),
            pltpu.VMEM((2, 8, 8, 129), jnp.float32),
            pltpu.SemaphoreType.DMA,
            pltpu.SemaphoreType.DMA,
            pltpu.SemaphoreType.DMA,
            pltpu.SemaphoreType.DMA,
            pltpu.SemaphoreType.DMA,
            pltpu.SemaphoreType.DMA,
        ],
        compiler_params=pltpu.CompilerParams(use_tc_tiling_on_sc=False,
                                             needs_layout_passes=False),
    )
    return f(story_t, table)


LCH = 40
NCH = L // LCH


def _attn_body(q_ref, gp_ref, dh_ref, len_ref, p_ref, soft_ref, logit_ref,
               pbuf, sem0, sem1):
    bt = pl.program_id(0)
    sems = (sem0, sem1)

    def hop_copy(h):
        return pltpu.make_async_copy(
            p_ref.at[h, :, :, bt, :, :], pbuf.at[h % 2], sems[h % 2])

    hop_copy(0).start()
    hop_copy(1).start()

    u = q_ref[...]
    lens = len_ref[...][0:1, :]
    gp = gp_ref[...]

    def mem_slice(h, c):
        lsl = pl.ds(c * LCH, LCH)
        lpos = lax.broadcasted_iota(jnp.int32, (LCH, 128), 0) + c * LCH
        mask = (lpos < lens).astype(jnp.float32)
        p = pbuf[h % 2, lsl, :, :, :].reshape(LCH, D, 128)
        return p + dh_ref[lsl, :, :] * mask[:, None, :]

    logits = None
    soft = None
    for h in range(NHOP):
        if h == 0:
            hop_copy(0).wait()
        parts = [jnp.sum(mem_slice(h, c) * u[None, :, :], axis=1)
                 for c in range(NCH)]
        logits = jnp.concatenate(parts, axis=0) * gp
        mx = jnp.max(logits, axis=0, keepdims=True)
        e = jnp.exp(logits - mx)
        soft = e / jnp.sum(e, axis=0, keepdims=True)
        if h < NHOP - 1:
            hop_copy(h + 1).wait()
            if h + 2 <= NHOP - 1:
                hop_copy(h + 2).start()
            w = soft * gp
            o = jnp.zeros((D, 128), jnp.float32)
            for c in range(NCH):
                wc = w[c * LCH:(c + 1) * LCH, None, :]
                o = o + jnp.sum(mem_slice(h + 1, c) * wc, axis=0)
            u = u + o
    soft_ref[...] = soft
    logit_ref[...] = logits


def _attn_tc(q_t, gp_t, dh_t, lens8, pooled6):
    grid = (NBT,)
    out_shape = [
        jax.ShapeDtypeStruct((L, B), jnp.float32),
        jax.ShapeDtypeStruct((L, B), jnp.float32),
    ]
    return pl.pallas_call(
        _attn_body,
        grid=grid,
        in_specs=[
            pl.BlockSpec((D, 128), lambda i: (0, i)),
            pl.BlockSpec((L, 128), lambda i: (0, i)),
            pl.BlockSpec((L, D, 128), lambda i: (0, 0, i)),
            pl.BlockSpec((8, 128), lambda i: (0, i)),
            pl.BlockSpec(memory_space=pl.ANY),
        ],
        out_specs=[
            pl.BlockSpec((L, 128), lambda i: (0, i)),
            pl.BlockSpec((L, 128), lambda i: (0, i)),
        ],
        out_shape=out_shape,
        scratch_shapes=[
            pltpu.VMEM((2, L, 8, 8, 128), jnp.float32),
            pltpu.SemaphoreType.DMA,
            pltpu.SemaphoreType.DMA,
        ],
    )(q_t, gp_t, dh_t, lens8, pooled6)


def kernel(query_vector, global_pointer, dh_outputs, CS, story, conv_len):
    table = CS[:NHOP].reshape(-1, D)
    story_t = jnp.transpose(story, (1, 2, 0)).astype(jnp.int32)
    dh_t = jnp.transpose(dh_outputs, (1, 2, 0))
    gp_t = global_pointer.T
    q_t = query_vector.T
    lens8 = jnp.broadcast_to(conv_len.reshape(1, B).astype(jnp.int32), (8, B))
    pooled6 = _pool_sc(story_t, table)
    soft_t, logits_t = _attn_tc(q_t, gp_t, dh_t, lens8, pooled6)
    return (soft_t.T, logits_t.T)


# speedup vs baseline: 3.7985x; 1.0202x over previous
import jax
import jax.numpy as jnp
from jax import lax
from jax.experimental import pallas as pl
from jax.experimental.pallas import tpu as pltpu
from jax.experimental.pallas import tpu_sc as plsc

VOCAB = 100000
D = 64
B = 1024
L = 200
M = 4
NHOP = 3

NBT = B // 128
NLG = 32 // NBT
LPW = L // NLG
N_UNITS = NHOP * LPW
G = 128


def _pool_body(story_hbm, table_hbm, out_hbm, idx_v, rows_v, pool_v,
               sem_i0, sem_i1, sem_g0, sem_g1, sem_w0, sem_w1):
    wid = lax.axis_index("s") * 2 + lax.axis_index("c")
    bt = wid % NBT
    l0 = (wid // NBT) * LPW
    b0 = bt * 128
    sem_i = (sem_i0, sem_i1)
    sem_g = (sem_g0, sem_g1)
    sem_w = (sem_w0, sem_w1)

    def unit_info(s):
        s = jnp.minimum(s, N_UNITS - 1)
        h = s // LPW
        l = l0 + s - h * LPW
        return h, l

    def idx_copy(s, buf):
        _, l = unit_info(s)
        return pltpu.make_async_copy(
            story_hbm.at[l, :, pl.ds(b0, 128)], idx_v.at[buf], sem_i[buf])

    def add_offs(s, buf):
        h, _ = unit_info(s)
        off = (h * VOCAB).astype(jnp.int32)
        for j in range(M):
            for k in range(G // 16):
                sl = pl.ds(k * 16, 16)
                idx_v[buf, j, sl] = idx_v[buf, j, sl] + off

    def gather_copies(buf):
        return [pltpu.make_async_copy(
                    table_hbm.at[idx_v.at[buf, j]],
                    rows_v.at[buf, pl.ds(j * G, G)], sem_g[buf])
                for j in range(M)]

    def write_copy(s, buf):
        h, l = unit_info(s)
        return pltpu.make_async_copy(
            pool_v.at[buf, :, :, pl.ds(0, 128)], out_hbm.at[h, l, :, bt],
            sem_w[buf])

    def transpose_sum(buf):
        lane = lax.iota(jnp.int32, 16)
        dtv = [(lane + g * 16) // 8 for g in range(D // 16)]
        d8v = [(lane + g * 16) % 8 for g in range(D // 16)]

        def rbody(bl, c):
            bv = jnp.full((16,), bl, jnp.int32)
            for g in range(D // 16):
                sl = pl.ds(g * 16, 16)
                s = (rows_v[buf, bl, sl] + rows_v[buf, G + bl, sl]
                     + rows_v[buf, 2 * G + bl, sl]
                     + rows_v[buf, 3 * G + bl, sl])
                plsc.store_scatter(pool_v.at[buf], [dtv[g], d8v[g], bv], s)
            return c
        lax.fori_loop(0, G, rbody, 0, unroll=4)

    idx_copy(0, 0).start()
    idx_copy(1, 1).start()
    idx_copy(0, 0).wait()
    add_offs(0, 0)
    for c in gather_copies(0):
        c.start()

    def phase(s, buf):
        nxt = 1 - buf
        idx_copy(s + 1, nxt).wait()
        add_offs(s + 1, nxt)
        for c in gather_copies(nxt):
            c.start()
        for c in gather_copies(buf):
            c.wait()
        idx_copy(s + 2, buf).start()

        @pl.when(s >= 2)
        def _():
            write_copy(s - 2, buf).wait()

        transpose_sum(buf)
        write_copy(s, buf).start()

    def loop_body(t, c):
        phase(2 * t, 0)
        phase(2 * t + 1, 1)
        return c
    lax.fori_loop(0, N_UNITS // 2, loop_body, 0)

    for c in gather_copies(0):
        c.wait()
    idx_copy(N_UNITS + 1, 1).wait()
    write_copy(N_UNITS - 2, 0).wait()
    write_copy(N_UNITS - 1, 1).wait()


def _pool_sc(story_t, table):
    f = pl.kernel(
        _pool_body,
        out_type=jax.ShapeDtypeStruct((NHOP, L, 8, NBT, 8, 128),
                                      jnp.float32),
        mesh=plsc.VectorSubcoreMesh(core_axis_name="c", subcore_axis_name="s",
                                    num_cores=2, num_subcores=16),
        scratch_types=[
            pltpu.VMEM((2, M, G), jnp.int32),
            pltpu.VMEM((2, M * G, D), jnp.float32),
            pltpu.VMEM((2, 8, 8, 129), jnp.float32),
            pltpu.SemaphoreType.DMA,
            pltpu.SemaphoreType.DMA,
            pltpu.SemaphoreType.DMA,
            pltpu.SemaphoreType.DMA,
            pltpu.SemaphoreType.DMA,
            pltpu.SemaphoreType.DMA,
        ],
        compiler_params=pltpu.CompilerParams(use_tc_tiling_on_sc=False,
                                             needs_layout_passes=False),
    )
    return f(story_t, table)


LCH = 40
NCH = L // LCH


def _attn_body(q_ref, gp_ref, dh_ref, len_ref, p_ref, soft_ref, logit_ref,
               pbuf, sems):
    bt = pl.program_id(0)

    def hop_copy(h, tile):
        slot = (h + tile) % 2
        return pltpu.make_async_copy(
            p_ref.at[h, :, :, tile, :, :], pbuf.at[slot], sems.at[slot])

    @pl.when(bt == 0)
    def _():
        hop_copy(0, bt).start()
    hop_copy(1, bt).start()

    u = q_ref[...]
    lens = len_ref[...][0:1, :]
    gp = gp_ref[...]

    def mem_slice(h, c):
        lsl = pl.ds(c * LCH, LCH)
        lpos = lax.broadcasted_iota(jnp.int32, (LCH, 128), 0) + c * LCH
        mask = (lpos < lens).astype(jnp.float32)
        p = pbuf[(h + bt) % 2, lsl, :, :, :].reshape(LCH, D, 128)
        return p + dh_ref[lsl, :, :] * mask[:, None, :]

    logits = None
    soft = None
    for h in range(NHOP):
        if h == 0:
            hop_copy(0, bt).wait()
        parts = [jnp.sum(mem_slice(h, c) * u[None, :, :], axis=1)
                 for c in range(NCH)]
        logits = jnp.concatenate(parts, axis=0) * gp
        mx = jnp.max(logits, axis=0, keepdims=True)
        e = jnp.exp(logits - mx)
        soft = e / jnp.sum(e, axis=0, keepdims=True)
        if h == 0:
            hop_copy(2, bt).start()
        if h == 1:
            @pl.when(bt + 1 < NBT)
            def _():
                hop_copy(0, bt + 1).start()
        if h < NHOP - 1:
            hop_copy(h + 1, bt).wait()
            w = soft * gp
            o = jnp.zeros((D, 128), jnp.float32)
            for c in range(NCH):
                wc = w[c * LCH:(c + 1) * LCH, None, :]
                o = o + jnp.sum(mem_slice(h + 1, c) * wc, axis=0)
            u = u + o
    soft_ref[...] = soft
    logit_ref[...] = logits


def _attn_tc(q_t, gp_t, dh_t, lens8, pooled6):
    grid = (NBT,)
    out_shape = [
        jax.ShapeDtypeStruct((L, B), jnp.float32),
        jax.ShapeDtypeStruct((L, B), jnp.float32),
    ]
    return pl.pallas_call(
        _attn_body,
        grid=grid,
        in_specs=[
            pl.BlockSpec((D, 128), lambda i: (0, i)),
            pl.BlockSpec((L, 128), lambda i: (0, i)),
            pl.BlockSpec((L, D, 128), lambda i: (0, 0, i)),
            pl.BlockSpec((8, 128), lambda i: (0, i)),
            pl.BlockSpec(memory_space=pl.ANY),
        ],
        out_specs=[
            pl.BlockSpec((L, 128), lambda i: (0, i)),
            pl.BlockSpec((L, 128), lambda i: (0, i)),
        ],
        out_shape=out_shape,
        scratch_shapes=[
            pltpu.VMEM((2, L, 8, 8, 128), jnp.float32),
            pltpu.SemaphoreType.DMA((2,)),
        ],
    )(q_t, gp_t, dh_t, lens8, pooled6)


def kernel(query_vector, global_pointer, dh_outputs, CS, story, conv_len):
    table = CS[:NHOP].reshape(-1, D)
    story_t = jnp.transpose(story, (1, 2, 0)).astype(jnp.int32)
    dh_t = jnp.transpose(dh_outputs, (1, 2, 0))
    gp_t = global_pointer.T
    q_t = query_vector.T
    lens8 = jnp.broadcast_to(conv_len.reshape(1, B).astype(jnp.int32), (8, B))
    pooled6 = _pool_sc(story_t, table)
    soft_t, logits_t = _attn_tc(q_t, gp_t, dh_t, lens8, pooled6)
    return (soft_t.T, logits_t.T)
